# baseline (speedup 1.0000x reference)
import functools

import jax
import jax.numpy as jnp
from jax import lax
from jax.experimental import pallas as pl
from jax.experimental.pallas import tpu as pltpu
from jax.experimental.pallas import tpu_sc as plsc

_NUM_CORES = 2
_NUM_SUBCORES = 16
_LANES = 16


def _sc_masked_segment_sum(ids_flat, msk_flat, embed, seg, n_codes, h):
    nw = _NUM_CORES * _NUM_SUBCORES
    seg_per_w = seg // nw
    ch = 4
    chunks = seg_per_w // ch
    ids_per_chunk = ch * n_codes

    mesh = plsc.VectorSubcoreMesh(core_axis_name="c", subcore_axis_name="s",
                                  num_cores=_NUM_CORES,
                                  num_subcores=_NUM_SUBCORES)

    @functools.partial(
        pl.kernel,
        out_type=jax.ShapeDtypeStruct((seg, h), jnp.float32),
        mesh=mesh,
        scratch_types=[
            pltpu.VMEM((ids_per_chunk,), jnp.int32),
            pltpu.VMEM((ids_per_chunk,), jnp.int32),
            pltpu.VMEM((ids_per_chunk, h), jnp.float32),
            pltpu.VMEM((ch, h), jnp.float32),
            pltpu.SemaphoreType.DMA,
        ],
    )
    def sc_kernel(ids_hbm, msk_hbm, embed_hbm, out_hbm, ids_v, msk_v, rows_v,
                  acc_v, sem):
        wid = lax.axis_index("s") * _NUM_CORES + lax.axis_index("c")

        def chunk_body(c, carry):
            base_seg = wid * seg_per_w + c * ch
            base_el = base_seg * n_codes
            pltpu.sync_copy(ids_hbm.at[pl.ds(base_el, ids_per_chunk)], ids_v)
            pltpu.sync_copy(msk_hbm.at[pl.ds(base_el, ids_per_chunk)], msk_v)
            for j in range(ids_per_chunk // _LANES):
                sl = pl.ds(j * _LANES, _LANES)
                ids_v[sl] = jnp.where(msk_v[sl] != 0, 0, ids_v[sl])
            pltpu.async_copy(embed_hbm.at[ids_v], rows_v, sem).wait()

            def seg_body(s, carry2):
                r0 = s * n_codes
                accs = [rows_v[r0, pl.ds(cc * _LANES, _LANES)]
                        for cc in range(h // _LANES)]
                for n in range(1, n_codes):
                    for cc in range(h // _LANES):
                        accs[cc] = accs[cc] + rows_v[r0 + n,
                                                     pl.ds(cc * _LANES, _LANES)]
                for cc in range(h // _LANES):
                    acc_v[s, pl.ds(cc * _LANES, _LANES)] = accs[cc]
                return carry2

            lax.fori_loop(0, ch, seg_body, 0)
            pltpu.sync_copy(acc_v, out_hbm.at[pl.ds(base_seg, ch)])
            return carry

        lax.fori_loop(0, chunks, chunk_body, 0)

    return sc_kernel(ids_flat, msk_flat, embed)


def _tc_grasp(pooled, maskf, proj_W, proj_b, W_ih, W_hh, b_ih, b_hh,
              centers, head_W, head_b, batch, v, n_codes, h, out_dim):
    seg = v * batch
    f32 = jnp.float32

    def tc_kernel(pooled_ref, maskf_ref, projW_ref, projb_ref, Wih_ref,
                  Whh_ref, bih_ref, bhh_ref, cent_ref, headW_ref, headb_ref,
                  out_ref, x_ref, mvis_ref):
        msum = jnp.sum(maskf_ref[...], axis=1, keepdims=True)
        cnt = jnp.float32(n_codes) - msum
        valid = (cnt > 0.0).astype(f32)
        denom = jnp.where(cnt > 0.0, cnt, 1.0)
        means = pooled_ref[...] * (1.0 / denom)
        x = lax.dot_general(means, projW_ref[...], (((1,), (0,)), ((), ())),
                            preferred_element_type=f32)
        x_ref[...] = x + valid * projb_ref[...]
        mvis_ref[...] = jnp.broadcast_to(valid, (seg, h))

        Wih = Wih_ref[...]
        Whh = Whh_ref[...]
        bih = bih_ref[...]
        bhh = bhh_ref[...]

        def step(t, hcar):
            xt = x_ref[pl.ds(t * batch, batch), :]
            mt = mvis_ref[pl.ds(t * batch, batch), :]
            gi = lax.dot_general(xt, Wih, (((1,), (0,)), ((), ())),
                                 preferred_element_type=f32) + bih
            gh = lax.dot_general(hcar, Whh, (((1,), (0,)), ((), ())),
                                 preferred_element_type=f32) + bhh
            r = jax.nn.sigmoid(gi[:, :h] + gh[:, :h])
            z = jax.nn.sigmoid(gi[:, h:2 * h] + gh[:, h:2 * h])
            nn = jnp.tanh(gi[:, 2 * h:] + r * gh[:, 2 * h:])
            h_new = (1.0 - z) * nn + z * hcar
            return mt * h_new + (1.0 - mt) * hcar

        hlast = lax.fori_loop(0, v, step, jnp.zeros((batch, h), f32))
        logits = lax.dot_general(hlast, cent_ref[...], (((1,), (1,)), ((), ())),
                                 preferred_element_type=f32)
        assign = jax.nn.softmax(logits, axis=-1)
        hout = hlast + lax.dot_general(assign, cent_ref[...],
                                       (((1,), (0,)), ((), ())),
                                       preferred_element_type=f32)
        out_ref[...] = lax.dot_general(hout, headW_ref[...],
                                       (((1,), (0,)), ((), ())),
                                       preferred_element_type=f32) + headb_ref[...]

    return pl.pallas_call(
        tc_kernel,
        out_shape=jax.ShapeDtypeStruct((batch, out_dim), f32),
        scratch_shapes=[
            pltpu.VMEM((seg, h), f32),
            pltpu.VMEM((seg, h), f32),
        ],
    )(pooled, maskf, proj_W, proj_b, W_ih, W_hh, b_ih, b_hh, centers,
      head_W, head_b)


def kernel(node_ids, edge_idx, edge_attr, visit_times, attn_mask, embed,
           proj_W, proj_b, W_ih, W_hh, b_ih, b_hh, centers, head_W, head_b):
    batch, v, n_codes = node_ids.shape
    h = embed.shape[1]
    out_dim = head_W.shape[1]
    ids_vb = jnp.transpose(node_ids, (1, 0, 2)).reshape(-1).astype(jnp.int32)
    msk_vb = jnp.transpose(attn_mask, (1, 0, 2)).reshape(-1).astype(jnp.int32)
    pooled = _sc_masked_segment_sum(ids_vb, msk_vb, embed, v * batch, n_codes, h)
    maskf = jnp.transpose(attn_mask, (1, 0, 2)).reshape(v * batch,
                                                        n_codes).astype(jnp.float32)
    return _tc_grasp(pooled, maskf, proj_W, proj_b.reshape(1, h), W_ih, W_hh,
                     b_ih.reshape(1, -1), b_hh.reshape(1, -1), centers,
                     head_W, head_b.reshape(1, -1), batch, v, n_codes, h,
                     out_dim)


# speedup vs baseline: 1.0045x; 1.0045x over previous
import functools

import jax
import jax.numpy as jnp
from jax import lax
from jax.experimental import pallas as pl
from jax.experimental.pallas import tpu as pltpu
from jax.experimental.pallas import tpu_sc as plsc

_NUM_CORES = 2
_NUM_SUBCORES = 16
_LANES = 16


def _sc_masked_segment_sum(ids2d, msk2d, embed, seg, n_codes, h):
    nw = _NUM_CORES * _NUM_SUBCORES
    seg_per_w = seg // nw
    ch = 4
    chunks = seg_per_w // ch
    ids_per_chunk = ch * n_codes

    mesh = plsc.VectorSubcoreMesh(core_axis_name="c", subcore_axis_name="s",
                                  num_cores=_NUM_CORES,
                                  num_subcores=_NUM_SUBCORES)

    @functools.partial(
        pl.kernel,
        out_type=jax.ShapeDtypeStruct((seg, h), jnp.float32),
        mesh=mesh,
        scratch_types=[
            pltpu.VMEM((chunks, ids_per_chunk), jnp.int32),
            pltpu.VMEM((chunks, ids_per_chunk), jnp.int32),
            pltpu.VMEM((2, ids_per_chunk, h), jnp.float32),
            pltpu.VMEM((seg_per_w, h), jnp.float32),
            pltpu.SemaphoreType.DMA,
            pltpu.SemaphoreType.DMA,
        ],
    )
    def sc_kernel(ids_hbm, msk_hbm, embed_hbm, out_hbm, ids_v, msk_v, rows_v,
                  out_v, sem0, sem1):
        wid = lax.axis_index("s") * _NUM_CORES + lax.axis_index("c")

        pltpu.sync_copy(ids_hbm.at[wid], ids_v)
        pltpu.sync_copy(msk_hbm.at[wid], msk_v)

        def mask_body(r, carry):
            for j in range(ids_per_chunk // _LANES):
                sl = pl.ds(j * _LANES, _LANES)
                ids_v[r, sl] = jnp.where(msk_v[r, sl] != 0, 0, ids_v[r, sl])
            return carry

        lax.fori_loop(0, chunks, mask_body, 0)

        def start_gather(c, buf, sem):
            pltpu.async_copy(embed_hbm.at[ids_v.at[c]], rows_v.at[buf], sem)

        def wait_gather(c, buf, sem):
            pltpu.make_async_copy(embed_hbm.at[ids_v.at[c]], rows_v.at[buf],
                                  sem).wait()

        def reduce_chunk(c, buf):
            def seg_body(s, carry):
                r0 = s * n_codes
                accs = [rows_v[buf, r0, pl.ds(cc * _LANES, _LANES)]
                        for cc in range(h // _LANES)]
                for n in range(1, n_codes):
                    for cc in range(h // _LANES):
                        accs[cc] = accs[cc] + rows_v[buf, r0 + n,
                                                     pl.ds(cc * _LANES, _LANES)]
                for cc in range(h // _LANES):
                    out_v[c * ch + s, pl.ds(cc * _LANES, _LANES)] = accs[cc]
                return carry

            lax.fori_loop(0, ch, seg_body, 0)

        start_gather(0, 0, sem0)

        def pipe_body(half, carry):
            c0 = half * 2
            start_gather(c0 + 1, 1, sem1)
            wait_gather(c0, 0, sem0)
            reduce_chunk(c0, 0)

            @pl.when(c0 + 2 < chunks)
            def _():
                start_gather(c0 + 2, 0, sem0)

            wait_gather(c0 + 1, 1, sem1)
            reduce_chunk(c0 + 1, 1)
            return carry

        lax.fori_loop(0, chunks // 2, pipe_body, 0)
        pltpu.sync_copy(out_v, out_hbm.at[pl.ds(wid * seg_per_w, seg_per_w)])

    return sc_kernel(ids2d, msk2d, embed)


def _tc_grasp(pooled, maskf, proj_W, proj_b, W_ih, W_hh, b_ih, b_hh,
              centers, head_W, head_b, batch, v, n_codes, h, out_dim):
    seg = v * batch
    f32 = jnp.float32

    def tc_kernel(pooled_ref, maskf_ref, projW_ref, projb_ref, Wih_ref,
                  Whh_ref, bih_ref, bhh_ref, cent_ref, headW_ref, headb_ref,
                  out_ref, x_ref, mvis_ref):
        msum = jnp.sum(maskf_ref[...].astype(f32), axis=1,
                       keepdims=True)
        cnt = jnp.float32(n_codes) - msum
        valid = (cnt > 0.0).astype(f32)
        denom = jnp.where(cnt > 0.0, cnt, 1.0)
        means = pooled_ref[...] * (1.0 / denom)
        x = lax.dot_general(means, projW_ref[...], (((1,), (0,)), ((), ())),
                            preferred_element_type=f32)
        x_ref[...] = x + valid * projb_ref[...]
        mvis_ref[...] = jnp.broadcast_to(valid, (seg, h))

        Wih = Wih_ref[...]
        Whh = Whh_ref[...]
        bih = bih_ref[...]
        bhh = bhh_ref[...]

        def step(t, hcar):
            xt = x_ref[pl.ds(t * batch, batch), :]
            mt = mvis_ref[pl.ds(t * batch, batch), :]
            gi = lax.dot_general(xt, Wih, (((1,), (0,)), ((), ())),
                                 preferred_element_type=f32) + bih
            gh = lax.dot_general(hcar, Whh, (((1,), (0,)), ((), ())),
                                 preferred_element_type=f32) + bhh
            r = jax.nn.sigmoid(gi[:, :h] + gh[:, :h])
            z = jax.nn.sigmoid(gi[:, h:2 * h] + gh[:, h:2 * h])
            nn = jnp.tanh(gi[:, 2 * h:] + r * gh[:, 2 * h:])
            h_new = (1.0 - z) * nn + z * hcar
            return mt * h_new + (1.0 - mt) * hcar

        hlast = lax.fori_loop(0, v, step, jnp.zeros((batch, h), f32))
        logits = lax.dot_general(hlast, cent_ref[...], (((1,), (1,)), ((), ())),
                                 preferred_element_type=f32)
        assign = jax.nn.softmax(logits, axis=-1)
        hout = hlast + lax.dot_general(assign, cent_ref[...],
                                       (((1,), (0,)), ((), ())),
                                       preferred_element_type=f32)
        out_ref[...] = lax.dot_general(hout, headW_ref[...],
                                       (((1,), (0,)), ((), ())),
                                       preferred_element_type=f32) + headb_ref[...]

    return pl.pallas_call(
        tc_kernel,
        out_shape=jax.ShapeDtypeStruct((batch, out_dim), f32),
        scratch_shapes=[
            pltpu.VMEM((seg, h), f32),
            pltpu.VMEM((seg, h), f32),
        ],
    )(pooled, maskf, proj_W, proj_b, W_ih, W_hh, b_ih, b_hh, centers,
      head_W, head_b)


def kernel(node_ids, edge_idx, edge_attr, visit_times, attn_mask, embed,
           proj_W, proj_b, W_ih, W_hh, b_ih, b_hh, centers, head_W, head_b):
    batch, v, n_codes = node_ids.shape
    h = embed.shape[1]
    out_dim = head_W.shape[1]
    nw = _NUM_CORES * _NUM_SUBCORES
    chunks = (v * batch) // (nw * 4)
    ids_vb = jnp.transpose(node_ids, (1, 0, 2)).reshape(
        nw, chunks, 4 * n_codes).astype(jnp.int32)
    msk_vb = jnp.transpose(attn_mask, (1, 0, 2)).reshape(
        nw, chunks, 4 * n_codes).astype(jnp.int32)
    pooled = _sc_masked_segment_sum(ids_vb, msk_vb, embed, v * batch, n_codes, h)
    maski = msk_vb.reshape(v * batch, n_codes)
    return _tc_grasp(pooled, maski, proj_W, proj_b.reshape(1, h), W_ih, W_hh,
                     b_ih.reshape(1, -1), b_hh.reshape(1, -1), centers,
                     head_W, head_b.reshape(1, -1), batch, v, n_codes, h,
                     out_dim)


# speedup vs baseline: 1.0427x; 1.0380x over previous
import functools

import jax
import jax.numpy as jnp
from jax import lax
from jax.experimental import pallas as pl
from jax.experimental.pallas import tpu as pltpu
from jax.experimental.pallas import tpu_sc as plsc

_NUM_CORES = 2
_NUM_SUBCORES = 16
_LANES = 16


def _sc_masked_segment_sum(ids2d, msk2d, embed, seg, n_codes, h):
    nw = _NUM_CORES * _NUM_SUBCORES
    seg_per_w = seg // nw
    ch = 5
    chunks = seg_per_w // ch
    ids_per_chunk = ch * n_codes
    nbuf = 4

    mesh = plsc.VectorSubcoreMesh(core_axis_name="c", subcore_axis_name="s",
                                  num_cores=_NUM_CORES,
                                  num_subcores=_NUM_SUBCORES)

    @functools.partial(
        pl.kernel,
        out_type=jax.ShapeDtypeStruct((seg, h), jnp.float32),
        mesh=mesh,
        scratch_types=[
            pltpu.VMEM((chunks, ids_per_chunk), jnp.int32),
            pltpu.VMEM((chunks, ids_per_chunk), jnp.int32),
            pltpu.VMEM((nbuf, ids_per_chunk, h), jnp.float32),
            pltpu.VMEM((seg_per_w, h), jnp.float32),
        ] + [pltpu.SemaphoreType.DMA] * nbuf,
    )
    def sc_kernel(ids_hbm, msk_hbm, embed_hbm, out_hbm, ids_v, msk_v, rows_v,
                  out_v, *sems):
        wid = lax.axis_index("s") * _NUM_CORES + lax.axis_index("c")

        pltpu.sync_copy(ids_hbm.at[wid], ids_v)
        pltpu.sync_copy(msk_hbm.at[wid], msk_v)

        def mask_body(r, carry):
            for j in range(ids_per_chunk // _LANES):
                sl = pl.ds(j * _LANES, _LANES)
                ids_v[r, sl] = jnp.where(msk_v[r, sl] != 0, 0, ids_v[r, sl])
            return carry

        lax.fori_loop(0, chunks, mask_body, 0)

        def start_gather(c, buf):
            pltpu.async_copy(embed_hbm.at[ids_v.at[c]], rows_v.at[buf],
                             sems[buf])

        def wait_gather(c, buf):
            pltpu.make_async_copy(embed_hbm.at[ids_v.at[c]], rows_v.at[buf],
                                  sems[buf]).wait()

        def reduce_chunk(c, buf):
            def seg_body(s, carry):
                r0 = s * n_codes
                accs = [rows_v[buf, r0, pl.ds(cc * _LANES, _LANES)]
                        for cc in range(h // _LANES)]
                for n in range(1, n_codes):
                    for cc in range(h // _LANES):
                        accs[cc] = accs[cc] + rows_v[buf, r0 + n,
                                                     pl.ds(cc * _LANES, _LANES)]
                for cc in range(h // _LANES):
                    out_v[c * ch + s, pl.ds(cc * _LANES, _LANES)] = accs[cc]
                return carry

            lax.fori_loop(0, ch, seg_body, 0)

        for b in range(nbuf - 1):
            start_gather(b, b)

        def pipe_body(g, carry):
            c0 = g * nbuf
            for b in range(nbuf):
                c = c0 + b

                @pl.when(c + nbuf - 1 < chunks)
                def _():
                    start_gather(c + nbuf - 1, (b + nbuf - 1) % nbuf)

                wait_gather(c, b)
                reduce_chunk(c, b)
            return carry

        lax.fori_loop(0, chunks // nbuf, pipe_body, 0)
        pltpu.sync_copy(out_v, out_hbm.at[pl.ds(wid * seg_per_w, seg_per_w)])

    return sc_kernel(ids2d, msk2d, embed)


def _tc_grasp(pooled, maskf, proj_W, proj_b, W_ih, W_hh, b_ih, b_hh,
              centers, head_W, head_b, batch, v, n_codes, h, out_dim):
    seg = v * batch
    f32 = jnp.float32

    def tc_kernel(pooled_ref, maskf_ref, projW_ref, projb_ref, Wih_ref,
                  Whh_ref, bih_ref, bhh_ref, cent_ref, headW_ref, headb_ref,
                  out_ref, x_ref, mvis_ref):
        msum = jnp.sum(maskf_ref[...].astype(f32), axis=1,
                       keepdims=True)
        cnt = jnp.float32(n_codes) - msum
        valid = (cnt > 0.0).astype(f32)
        denom = jnp.where(cnt > 0.0, cnt, 1.0)
        means = pooled_ref[...] * (1.0 / denom)
        x = lax.dot_general(means, projW_ref[...], (((1,), (0,)), ((), ())),
                            preferred_element_type=f32)
        x_ref[...] = x + valid * projb_ref[...]
        mvis_ref[...] = jnp.broadcast_to(valid, (seg, h))

        Wih = Wih_ref[...]
        Whh = Whh_ref[...]
        bih = bih_ref[...]
        bhh = bhh_ref[...]

        def step(t, hcar):
            xt = x_ref[pl.ds(t * batch, batch), :]
            mt = mvis_ref[pl.ds(t * batch, batch), :]
            gi = lax.dot_general(xt, Wih, (((1,), (0,)), ((), ())),
                                 preferred_element_type=f32) + bih
            gh = lax.dot_general(hcar, Whh, (((1,), (0,)), ((), ())),
                                 preferred_element_type=f32) + bhh
            r = jax.nn.sigmoid(gi[:, :h] + gh[:, :h])
            z = jax.nn.sigmoid(gi[:, h:2 * h] + gh[:, h:2 * h])
            nn = jnp.tanh(gi[:, 2 * h:] + r * gh[:, 2 * h:])
            h_new = (1.0 - z) * nn + z * hcar
            return mt * h_new + (1.0 - mt) * hcar

        hlast = lax.fori_loop(0, v, step, jnp.zeros((batch, h), f32))
        logits = lax.dot_general(hlast, cent_ref[...], (((1,), (1,)), ((), ())),
                                 preferred_element_type=f32)
        assign = jax.nn.softmax(logits, axis=-1)
        hout = hlast + lax.dot_general(assign, cent_ref[...],
                                       (((1,), (0,)), ((), ())),
                                       preferred_element_type=f32)
        out_ref[...] = lax.dot_general(hout, headW_ref[...],
                                       (((1,), (0,)), ((), ())),
                                       preferred_element_type=f32) + headb_ref[...]

    return pl.pallas_call(
        tc_kernel,
        out_shape=jax.ShapeDtypeStruct((batch, out_dim), f32),
        scratch_shapes=[
            pltpu.VMEM((seg, h), f32),
            pltpu.VMEM((seg, h), f32),
        ],
    )(pooled, maskf, proj_W, proj_b, W_ih, W_hh, b_ih, b_hh, centers,
      head_W, head_b)


def kernel(node_ids, edge_idx, edge_attr, visit_times, attn_mask, embed,
           proj_W, proj_b, W_ih, W_hh, b_ih, b_hh, centers, head_W, head_b):
    batch, v, n_codes = node_ids.shape
    h = embed.shape[1]
    out_dim = head_W.shape[1]
    nw = _NUM_CORES * _NUM_SUBCORES
    ch = 5
    chunks = (v * batch) // (nw * ch)
    ids_vb = jnp.transpose(node_ids, (1, 0, 2)).reshape(
        nw, chunks, ch * n_codes).astype(jnp.int32)
    msk_vb = jnp.transpose(attn_mask, (1, 0, 2)).reshape(
        nw, chunks, ch * n_codes).astype(jnp.int32)
    pooled = _sc_masked_segment_sum(ids_vb, msk_vb, embed, v * batch, n_codes, h)
    maski = msk_vb.reshape(v * batch, n_codes)
    return _tc_grasp(pooled, maski, proj_W, proj_b.reshape(1, h), W_ih, W_hh,
                     b_ih.reshape(1, -1), b_hh.reshape(1, -1), centers,
                     head_W, head_b.reshape(1, -1), batch, v, n_codes, h,
                     out_dim)


# speedup vs baseline: 32.7767x; 31.4336x over previous
import functools

import jax
import jax.numpy as jnp
from jax import lax
from jax.experimental import pallas as pl
from jax.experimental.pallas import tpu as pltpu
from jax.experimental.pallas import tpu_sc as plsc

_NUM_CORES = 2
_NUM_SUBCORES = 16
_LANES = 16


_PAD_ROWS = 4096


def _sc_masked_segment_sum(ids2d, msk2d, embed, seg, n_codes, h, pad_base):
    nw = _NUM_CORES * _NUM_SUBCORES
    seg_per_w = seg // nw
    ch = 5
    chunks = seg_per_w // ch
    ids_per_chunk = ch * n_codes
    nbuf = 4

    mesh = plsc.VectorSubcoreMesh(core_axis_name="c", subcore_axis_name="s",
                                  num_cores=_NUM_CORES,
                                  num_subcores=_NUM_SUBCORES)

    @functools.partial(
        pl.kernel,
        out_type=jax.ShapeDtypeStruct((seg, h), jnp.float32),
        mesh=mesh,
        scratch_types=[
            pltpu.VMEM((chunks, ids_per_chunk), jnp.int32),
            pltpu.VMEM((chunks, ids_per_chunk), jnp.int32),
            pltpu.VMEM((nbuf, ids_per_chunk, h), jnp.float32),
            pltpu.VMEM((seg_per_w, h), jnp.float32),
        ] + [pltpu.SemaphoreType.DMA] * nbuf,
    )
    def sc_kernel(ids_hbm, msk_hbm, embed_hbm, out_hbm, ids_v, msk_v, rows_v,
                  out_v, *sems):
        wid = lax.axis_index("s") * _NUM_CORES + lax.axis_index("c")

        pltpu.sync_copy(ids_hbm.at[wid], ids_v)
        pltpu.sync_copy(msk_hbm.at[wid], msk_v)

        def mask_body(r, carry):
            for j in range(ids_per_chunk // _LANES):
                sl = pl.ds(j * _LANES, _LANES)
                ids = ids_v[r, sl]
                pad = pad_base + (ids & (_PAD_ROWS - 1))
                ids_v[r, sl] = jnp.where(msk_v[r, sl] != 0, pad, ids)
            return carry

        lax.fori_loop(0, chunks, mask_body, 0)

        def start_gather(c, buf):
            pltpu.async_copy(embed_hbm.at[ids_v.at[c]], rows_v.at[buf],
                             sems[buf])

        def wait_gather(c, buf):
            pltpu.make_async_copy(embed_hbm.at[ids_v.at[c]], rows_v.at[buf],
                                  sems[buf]).wait()

        def reduce_chunk(c, buf):
            def seg_body(s, carry):
                r0 = s * n_codes
                accs = [rows_v[buf, r0, pl.ds(cc * _LANES, _LANES)]
                        for cc in range(h // _LANES)]
                for n in range(1, n_codes):
                    for cc in range(h // _LANES):
                        accs[cc] = accs[cc] + rows_v[buf, r0 + n,
                                                     pl.ds(cc * _LANES, _LANES)]
                for cc in range(h // _LANES):
                    out_v[c * ch + s, pl.ds(cc * _LANES, _LANES)] = accs[cc]
                return carry

            lax.fori_loop(0, ch, seg_body, 0)

        for b in range(nbuf - 1):
            start_gather(b, b)

        def pipe_body(g, carry):
            c0 = g * nbuf
            for b in range(nbuf):
                c = c0 + b

                @pl.when(c + nbuf - 1 < chunks)
                def _():
                    start_gather(c + nbuf - 1, (b + nbuf - 1) % nbuf)

                wait_gather(c, b)
                reduce_chunk(c, b)
            return carry

        lax.fori_loop(0, chunks // nbuf, pipe_body, 0)
        pltpu.sync_copy(out_v, out_hbm.at[pl.ds(wid * seg_per_w, seg_per_w)])

    return sc_kernel(ids2d, msk2d, embed)


def _tc_grasp(pooled, maskf, proj_W, proj_b, W_ih, W_hh, b_ih, b_hh,
              centers, head_W, head_b, batch, v, n_codes, h, out_dim):
    seg = v * batch
    f32 = jnp.float32

    def tc_kernel(pooled_ref, maskf_ref, projW_ref, projb_ref, Wih_ref,
                  Whh_ref, bih_ref, bhh_ref, cent_ref, headW_ref, headb_ref,
                  out_ref, x_ref, mvis_ref):
        msum = jnp.sum(maskf_ref[...].astype(f32), axis=1,
                       keepdims=True)
        cnt = jnp.float32(n_codes) - msum
        valid = (cnt > 0.0).astype(f32)
        denom = jnp.where(cnt > 0.0, cnt, 1.0)
        means = pooled_ref[...] * (1.0 / denom)
        x = lax.dot_general(means, projW_ref[...], (((1,), (0,)), ((), ())),
                            preferred_element_type=f32)
        x_ref[...] = x + valid * projb_ref[...]
        mvis_ref[...] = jnp.broadcast_to(valid, (seg, h))

        Wih = Wih_ref[...]
        Whh = Whh_ref[...]
        bih = bih_ref[...]
        bhh = bhh_ref[...]

        def step(t, hcar):
            xt = x_ref[pl.ds(t * batch, batch), :]
            mt = mvis_ref[pl.ds(t * batch, batch), :]
            gi = lax.dot_general(xt, Wih, (((1,), (0,)), ((), ())),
                                 preferred_element_type=f32) + bih
            gh = lax.dot_general(hcar, Whh, (((1,), (0,)), ((), ())),
                                 preferred_element_type=f32) + bhh
            r = jax.nn.sigmoid(gi[:, :h] + gh[:, :h])
            z = jax.nn.sigmoid(gi[:, h:2 * h] + gh[:, h:2 * h])
            nn = jnp.tanh(gi[:, 2 * h:] + r * gh[:, 2 * h:])
            h_new = (1.0 - z) * nn + z * hcar
            return mt * h_new + (1.0 - mt) * hcar

        hlast = lax.fori_loop(0, v, step, jnp.zeros((batch, h), f32))
        logits = lax.dot_general(hlast, cent_ref[...], (((1,), (1,)), ((), ())),
                                 preferred_element_type=f32)
        assign = jax.nn.softmax(logits, axis=-1)
        hout = hlast + lax.dot_general(assign, cent_ref[...],
                                       (((1,), (0,)), ((), ())),
                                       preferred_element_type=f32)
        out_ref[...] = lax.dot_general(hout, headW_ref[...],
                                       (((1,), (0,)), ((), ())),
                                       preferred_element_type=f32) + headb_ref[...]

    return pl.pallas_call(
        tc_kernel,
        out_shape=jax.ShapeDtypeStruct((batch, out_dim), f32),
        scratch_shapes=[
            pltpu.VMEM((seg, h), f32),
            pltpu.VMEM((seg, h), f32),
        ],
    )(pooled, maskf, proj_W, proj_b, W_ih, W_hh, b_ih, b_hh, centers,
      head_W, head_b)


def kernel(node_ids, edge_idx, edge_attr, visit_times, attn_mask, embed,
           proj_W, proj_b, W_ih, W_hh, b_ih, b_hh, centers, head_W, head_b):
    batch, v, n_codes = node_ids.shape
    h = embed.shape[1]
    out_dim = head_W.shape[1]
    nw = _NUM_CORES * _NUM_SUBCORES
    ch = 5
    chunks = (v * batch) // (nw * ch)
    ids_vb = jnp.transpose(node_ids, (1, 0, 2)).reshape(
        nw, chunks, ch * n_codes).astype(jnp.int32)
    msk_vb = jnp.transpose(attn_mask, (1, 0, 2)).reshape(
        nw, chunks, ch * n_codes).astype(jnp.int32)
    embed_pad = jnp.pad(embed, ((0, _PAD_ROWS), (0, 0)))
    pooled = _sc_masked_segment_sum(ids_vb, msk_vb, embed_pad, v * batch,
                                    n_codes, h, embed.shape[0])
    maski = msk_vb.reshape(v * batch, n_codes)
    return _tc_grasp(pooled, maski, proj_W, proj_b.reshape(1, h), W_ih, W_hh,
                     b_ih.reshape(1, -1), b_hh.reshape(1, -1), centers,
                     head_W, head_b.reshape(1, -1), batch, v, n_codes, h,
                     out_dim)
